# Initial kernel scaffold; baseline (speedup 1.0000x reference)
import jax
import jax.numpy as jnp
from jax.experimental import pallas as pl


def kernel(x, adj, W1, b1, W2, b2, W3, b3, W4, b4, Wf, bf):
    raise NotImplementedError("write your pallas kernel here")


# speedup vs baseline: 1.1815x; 1.1815x over previous
import jax
import jax.numpy as jnp
from jax.experimental import pallas as pl

_B, _N, _NFEAT, _NHID = 8, 512, 256, 64


def _gcn_body(x_ref, adj_ref, w1_ref, b1_ref, w2_ref, b2_ref, w3_ref,
              b3_ref, w4_ref, b4_ref, wf_ref, bf_ref, out_ref):
    f32 = jnp.float32
    a = adj_ref[0]
    h = x_ref[0]
    for w_ref, b_ref in ((w1_ref, b1_ref), (w2_ref, b2_ref), (w3_ref, b3_ref)):
        s = jnp.dot(h, w_ref[...], preferred_element_type=f32)
        h = jnp.maximum(jnp.dot(a, s, preferred_element_type=f32)
                        + b_ref[...], 0.0)
    v = jnp.dot(a[_N - 1:_N, :], h, preferred_element_type=f32)
    h4 = jnp.maximum(jnp.dot(v, w4_ref[...], preferred_element_type=f32)
                     + b4_ref[...], 0.0)
    out_ref[0] = jnp.dot(h4, wf_ref[...], preferred_element_type=f32) \
        + bf_ref[...]


def kernel(x, adj, W1, b1, W2, b2, W3, b3, W4, b4, Wf, bf):
    wspec = lambda r, c: pl.BlockSpec((r, c), lambda b: (0, 0))
    out = pl.pallas_call(
        _gcn_body,
        grid=(_B,),
        in_specs=[
            pl.BlockSpec((1, _N, _NFEAT), lambda b: (b, 0, 0)),
            pl.BlockSpec((1, _N, _N), lambda b: (b, 0, 0)),
            wspec(_NFEAT, _NHID), wspec(1, _NHID),
            wspec(_NHID, _NHID), wspec(1, _NHID),
            wspec(_NHID, _NHID), wspec(1, _NHID),
            wspec(_NHID, _NHID), wspec(1, _NHID),
            wspec(_NHID, 1), wspec(1, 1),
        ],
        out_specs=pl.BlockSpec((1, 1, 1), lambda b: (b, 0, 0)),
        out_shape=jax.ShapeDtypeStruct((_B, 1, 1), jnp.float32),
    )(x, adj,
      W1, b1.reshape(1, _NHID), W2, b2.reshape(1, _NHID),
      W3, b3.reshape(1, _NHID), W4, b4.reshape(1, _NHID),
      Wf, bf.reshape(1, 1))
    return out.reshape(_B, 1)
